# baseline (device time: 33790 ns/iter reference)
import jax
import jax.numpy as jnp
from jax import lax
from jax.experimental import pallas as pl
from jax.experimental.pallas import tpu as pltpu

N_DEV = 4
WINDOW = 128


def kernel(x, Wq, K_ext, V_ext, Wo):
    B, Sq, _ = x.shape
    _, Skv, Hl, Dh = K_ext.shape
    D_out = Wo.shape[1]
    HD = Hl * Dh

    def body(x_ref, wq_ref, k_ref, v_ref, wo_ref, out_ref,
             comm_ref, send_sems, recv_sems):
        my_pos = lax.axis_index("i")
        right = lax.rem(my_pos + 1, N_DEV)
        left = lax.rem(my_pos + N_DEV - 1, N_DEV)

        barrier_sem = pltpu.get_barrier_semaphore()
        for nbr in (left, right):
            pl.semaphore_signal(
                barrier_sem, inc=1,
                device_id=(nbr,), device_id_type=pl.DeviceIdType.MESH,
            )
        pl.semaphore_wait(barrier_sem, 2)

        rows = lax.broadcasted_iota(jnp.int32, (Sq, Skv), 0)
        cols = lax.broadcasted_iota(jnp.int32, (Sq, Skv), 1)
        mask = jnp.abs(rows - cols) <= WINDOW

        base = my_pos * HD
        for b in range(B):
            xb = x_ref[b]
            wq_my = wq_ref[:, pl.ds(base, HD)]
            qb = jnp.dot(xb, wq_my, preferred_element_type=jnp.float32)
            for h in range(Hl):
                qh = qb[:, h * Dh:(h + 1) * Dh]
                kh = k_ref[b, :, h, :]
                vh = v_ref[b, :, h, :]
                s = lax.dot_general(
                    qh, kh, (((1,), (1,)), ((), ())),
                    preferred_element_type=jnp.float32,
                ) * 0.125
                s = jnp.where(mask, s, -1e9)
                m = jnp.max(s, axis=1, keepdims=True)
                w = jnp.exp(s - m)
                w = w / jnp.sum(w, axis=1, keepdims=True)
                comm_ref[0, b, :, h * Dh:(h + 1) * Dh] = jnp.dot(
                    w, vh, preferred_element_type=jnp.float32
                )

        def contrib(slot):
            origin = lax.rem(my_pos + N_DEV - slot, N_DEV)
            wo_block = wo_ref[pl.ds(origin * HD, HD), :]
            for b in range(B):
                delta = jnp.dot(
                    comm_ref[slot, b], wo_block,
                    preferred_element_type=jnp.float32,
                )
                if slot == 0:
                    out_ref[b] = delta
                else:
                    out_ref[b] = out_ref[b] + delta

        rdmas = []
        for hop in range(N_DEV - 1):
            rdma = pltpu.make_async_remote_copy(
                src_ref=comm_ref.at[hop],
                dst_ref=comm_ref.at[hop + 1],
                send_sem=send_sems.at[hop],
                recv_sem=recv_sems.at[hop],
                device_id=(right,),
                device_id_type=pl.DeviceIdType.MESH,
            )
            rdma.start()
            rdmas.append(rdma)
            contrib(hop)
            rdma.wait_recv()
        contrib(N_DEV - 1)
        for rdma in rdmas:
            rdma.wait_send()

    return pl.pallas_call(
        body,
        out_shape=jax.ShapeDtypeStruct((B, Sq, D_out), jnp.float32),
        in_specs=[pl.BlockSpec(memory_space=pltpu.VMEM)] * 5,
        out_specs=pl.BlockSpec(memory_space=pltpu.VMEM),
        scratch_shapes=[
            pltpu.VMEM((N_DEV, B, Sq, HD), jnp.float32),
            pltpu.SemaphoreType.DMA((N_DEV - 1,)),
            pltpu.SemaphoreType.DMA((N_DEV - 1,)),
        ],
        compiler_params=pltpu.CompilerParams(collective_id=0),
    )(x, Wq, K_ext, V_ext, Wo)


# device time: 19974 ns/iter; 1.6917x vs baseline; 1.6917x over previous
import jax
import jax.numpy as jnp
from jax import lax
from jax.experimental import pallas as pl
from jax.experimental.pallas import tpu as pltpu

N_DEV = 4
WINDOW = 128
CDT = jnp.bfloat16


def kernel(x, Wq, K_ext, V_ext, Wo):
    B, Sq, _ = x.shape
    _, Skv, Hl, Dh = K_ext.shape
    D_out = Wo.shape[1]
    HD = Hl * Dh

    def body(x_ref, wq_ref, k_ref, v_ref, wo_ref, out_ref,
             comm_ref, send_sems, recv_sems):
        my_pos = lax.axis_index("i")
        right = lax.rem(my_pos + 1, N_DEV)
        left = lax.rem(my_pos + N_DEV - 1, N_DEV)

        barrier_sem = pltpu.get_barrier_semaphore()
        for nbr in (left, right):
            pl.semaphore_signal(
                barrier_sem, inc=1,
                device_id=(nbr,), device_id_type=pl.DeviceIdType.MESH,
            )
        pl.semaphore_wait(barrier_sem, 2)

        rows = lax.broadcasted_iota(jnp.int32, (Sq, Skv), 0)
        cols = lax.broadcasted_iota(jnp.int32, (Sq, Skv), 1)
        mask = jnp.abs(rows - cols) <= WINDOW

        base = my_pos * HD
        wq_my = (wq_ref[:, pl.ds(base, HD)] * 0.125).astype(CDT)
        for b in range(B):
            xb = x_ref[b].astype(CDT)
            qb = jnp.dot(xb, wq_my, preferred_element_type=jnp.float32)
            for h in range(Hl):
                qh = qb[:, h * Dh:(h + 1) * Dh].astype(CDT)
                kh = k_ref[b, :, h, :].astype(CDT)
                vh = v_ref[b, :, h, :].astype(CDT)
                s = lax.dot_general(
                    qh, kh, (((1,), (1,)), ((), ())),
                    preferred_element_type=jnp.float32,
                )
                w = jnp.exp(jnp.where(mask, s, -1e9))
                w = w / jnp.sum(w, axis=1, keepdims=True)
                comm_ref[0, b, :, h * Dh:(h + 1) * Dh] = jnp.dot(
                    w.astype(CDT), vh, preferred_element_type=jnp.float32
                ).astype(CDT)

        SLOT_ORIGIN_DELTA = {0: 0, 1: N_DEV - 1, 2: 1, 3: 2}

        def contrib(slot):
            origin = lax.rem(my_pos + SLOT_ORIGIN_DELTA[slot], N_DEV)
            wo_block = wo_ref[pl.ds(origin * HD, HD), :].astype(CDT)
            for b in range(B):
                delta = jnp.dot(
                    comm_ref[slot, b], wo_block,
                    preferred_element_type=jnp.float32,
                )
                if slot == 0:
                    out_ref[b] = delta
                else:
                    out_ref[b] = out_ref[b] + delta

        def copy(src_at, dst_at, sem_idx, target):
            return pltpu.make_async_remote_copy(
                src_ref=src_at, dst_ref=dst_at,
                send_sem=send_sems.at[sem_idx],
                recv_sem=recv_sems.at[sem_idx],
                device_id=(target,), device_id_type=pl.DeviceIdType.MESH,
            )

        p1_r = copy(comm_ref.at[0], comm_ref.at[1], 0, right)
        p1_l = copy(comm_ref.at[0], comm_ref.at[2], 1, left)
        p1_r.start()
        p1_l.start()
        contrib(0)

        p1_r.wait_recv()
        p2_r = copy(comm_ref.at[1, 0], comm_ref.at[3, 0], 2, right)
        p2_r.start()
        contrib(1)
        p1_l.wait_recv()
        p2_l = copy(comm_ref.at[2, 1], comm_ref.at[3, 1], 3, left)
        p2_l.start()
        contrib(2)
        p2_r.wait_recv()
        p2_l.wait_recv()
        contrib(3)
        for rdma in (p1_r, p1_l, p2_r, p2_l):
            rdma.wait_send()

    return pl.pallas_call(
        body,
        out_shape=jax.ShapeDtypeStruct((B, Sq, D_out), jnp.float32),
        in_specs=[pl.BlockSpec(memory_space=pltpu.VMEM)] * 5,
        out_specs=pl.BlockSpec(memory_space=pltpu.VMEM),
        scratch_shapes=[
            pltpu.VMEM((N_DEV, B, Sq, HD), CDT),
            pltpu.SemaphoreType.DMA((4,)),
            pltpu.SemaphoreType.DMA((4,)),
        ],
        compiler_params=pltpu.CompilerParams(collective_id=0),
    )(x, Wq, K_ext, V_ext, Wo)


# device time: 19072 ns/iter; 1.7717x vs baseline; 1.0473x over previous
import jax
import jax.numpy as jnp
from jax import lax
from jax.experimental import pallas as pl
from jax.experimental.pallas import tpu as pltpu

N_DEV = 4
WINDOW = 128
CDT = jnp.bfloat16


def kernel(x, Wq, K_ext, V_ext, Wo):
    B, Sq, _ = x.shape
    _, Skv, Hl, Dh = K_ext.shape
    D_out = Wo.shape[1]
    HD = Hl * Dh
    HALF = HD // 2

    def body(x_ref, wq_ref, k_ref, v_ref, wo_ref, out_ref,
             comm_ref, send_sems, recv_sems):
        my_pos = lax.axis_index("i")
        right = lax.rem(my_pos + 1, N_DEV)
        left = lax.rem(my_pos + N_DEV - 1, N_DEV)

        barrier_sem = pltpu.get_barrier_semaphore()
        for nbr in (left, right):
            pl.semaphore_signal(
                barrier_sem, inc=1,
                device_id=(nbr,), device_id_type=pl.DeviceIdType.MESH,
            )
        pl.semaphore_wait(barrier_sem, 2)

        rows = lax.broadcasted_iota(jnp.int32, (Sq, Skv), 0)
        cols = lax.broadcasted_iota(jnp.int32, (Sq, Skv), 1)
        mask = jnp.abs(rows - cols) <= WINDOW

        base = my_pos * HD
        wq_my = (wq_ref[:, pl.ds(base, HD)] * 0.125).astype(CDT)
        x_all = x_ref[...].reshape(B * Sq, -1).astype(CDT)
        q_all = jnp.dot(x_all, wq_my, preferred_element_type=jnp.float32)

        def attend(b, h):
            qh = q_all[b * Sq:(b + 1) * Sq, h * Dh:(h + 1) * Dh].astype(CDT)
            kh = k_ref[b, :, h, :].astype(CDT)
            vh = v_ref[b, :, h, :].astype(CDT)
            s = lax.dot_general(
                qh, kh, (((1,), (1,)), ((), ())),
                preferred_element_type=jnp.float32,
            )
            w = jnp.exp(jnp.where(mask, s, -1e9))
            recip = 1.0 / jnp.sum(w, axis=1, keepdims=True)
            ctx = jnp.dot(w.astype(CDT), vh,
                          preferred_element_type=jnp.float32)
            comm_ref[0, b, :, h * Dh:(h + 1) * Dh] = (ctx * recip).astype(CDT)

        SLOT_ORIGIN_DELTA = {0: 0, 1: N_DEV - 1, 2: 1, 3: 2}

        def contrib(slot):
            origin = lax.rem(my_pos + SLOT_ORIGIN_DELTA[slot], N_DEV)
            wo_block = wo_ref[pl.ds(origin * HD, HD), :].astype(CDT)
            delta = jnp.dot(
                comm_ref[slot].reshape(B * Sq, HD), wo_block,
                preferred_element_type=jnp.float32,
            ).reshape(B, Sq, D_out)
            if slot == 0:
                out_ref[...] = delta
            else:
                out_ref[...] = out_ref[...] + delta

        def copy(src_at, dst_at, sem_idx, target):
            return pltpu.make_async_remote_copy(
                src_ref=src_at, dst_ref=dst_at,
                send_sem=send_sems.at[sem_idx],
                recv_sem=recv_sems.at[sem_idx],
                device_id=(target,), device_id_type=pl.DeviceIdType.MESH,
            )

        for b in range(B):
            for h in range(Hl // 2):
                attend(b, h)
        loA = (slice(None), slice(None), pl.ds(0, HALF))
        p1 = [
            copy(comm_ref.at[(0, *loA)], comm_ref.at[(1, *loA)], 0, right),
            copy(comm_ref.at[(0, *loA)], comm_ref.at[(2, *loA)], 1, left),
        ]
        p1[0].start()
        p1[1].start()

        for b in range(B):
            for h in range(Hl // 2, Hl):
                attend(b, h)
        hiA = (slice(None), slice(None), pl.ds(HALF, HALF))
        p1 += [
            copy(comm_ref.at[(0, *hiA)], comm_ref.at[(1, *hiA)], 2, right),
            copy(comm_ref.at[(0, *hiA)], comm_ref.at[(2, *hiA)], 3, left),
        ]
        p1[2].start()
        p1[3].start()
        contrib(0)

        p1[0].wait_recv()
        p1[2].wait_recv()
        p2_r = copy(comm_ref.at[1, 0], comm_ref.at[3, 0], 4, right)
        p2_r.start()
        contrib(1)
        p1[1].wait_recv()
        p1[3].wait_recv()
        p2_l = copy(comm_ref.at[2, 1], comm_ref.at[3, 1], 5, left)
        p2_l.start()
        contrib(2)
        p2_r.wait_recv()
        p2_l.wait_recv()
        contrib(3)
        for rdma in p1 + [p2_r, p2_l]:
            rdma.wait_send()

    return pl.pallas_call(
        body,
        out_shape=jax.ShapeDtypeStruct((B, Sq, D_out), jnp.float32),
        in_specs=[pl.BlockSpec(memory_space=pltpu.VMEM)] * 5,
        out_specs=pl.BlockSpec(memory_space=pltpu.VMEM),
        scratch_shapes=[
            pltpu.VMEM((N_DEV, B, Sq, HD), CDT),
            pltpu.SemaphoreType.DMA((6,)),
            pltpu.SemaphoreType.DMA((6,)),
        ],
        compiler_params=pltpu.CompilerParams(collective_id=0),
    )(x, Wq, K_ext, V_ext, Wo)


# device time: 17290 ns/iter; 1.9543x vs baseline; 1.1031x over previous
import os

import jax
import jax.numpy as jnp
from jax import lax

ABLATE = os.environ.get("ABLATE", "")
from jax.experimental import pallas as pl
from jax.experimental.pallas import tpu as pltpu

N_DEV = 4
WINDOW = 128
CDT = jnp.bfloat16


def kernel(x, Wq, K_ext, V_ext, Wo):
    B, Sq, _ = x.shape
    _, Skv, Hl, Dh = K_ext.shape
    D_out = Wo.shape[1]
    HD = Hl * Dh
    HALF = HD // 2

    def body(x_ref, wq_ref, k_ref, v_ref, wo_ref, out_ref,
             comm_ref, send_sems, recv_sems):
        my_pos = lax.axis_index("i")
        right = lax.rem(my_pos + 1, N_DEV)
        left = lax.rem(my_pos + N_DEV - 1, N_DEV)

        barrier_sem = pltpu.get_barrier_semaphore()
        for nbr in (left, right):
            pl.semaphore_signal(
                barrier_sem, inc=1,
                device_id=(nbr,), device_id_type=pl.DeviceIdType.MESH,
            )
        pl.semaphore_wait(barrier_sem, 2)

        rows = lax.broadcasted_iota(jnp.int32, (Sq, Skv), 0)
        cols = lax.broadcasted_iota(jnp.int32, (Sq, Skv), 1)
        mask = jnp.abs(rows - cols) <= WINDOW

        base = my_pos * HD
        wq_my = (wq_ref[:, pl.ds(base, HD)] * 0.125).astype(CDT)
        x_all = x_ref[...].reshape(B * Sq, -1).astype(CDT)
        q_all = jnp.dot(x_all, wq_my, preferred_element_type=jnp.float32)

        def attend(b, h):
            qh = q_all[b * Sq:(b + 1) * Sq, h * Dh:(h + 1) * Dh].astype(CDT)
            if ABLATE == "noattn":
                comm_ref[0, b, :, h * Dh:(h + 1) * Dh] = qh
                return
            kh = k_ref[b, :, h, :].astype(CDT)
            vh = v_ref[b, :, h, :].astype(CDT)
            s = lax.dot_general(
                qh, kh, (((1,), (1,)), ((), ())),
                preferred_element_type=jnp.float32,
            )
            w = jnp.exp(jnp.where(mask, s, -1e9))
            recip = 1.0 / jnp.sum(w, axis=1, keepdims=True)
            ctx = jnp.dot(w.astype(CDT), vh,
                          preferred_element_type=jnp.float32)
            comm_ref[0, b, :, h * Dh:(h + 1) * Dh] = (ctx * recip).astype(CDT)

        SLOT_ORIGIN_DELTA = {0: 0, 1: N_DEV - 1, 2: 1, 3: 2}

        def contrib(slot):
            if ABLATE == "nocontrib" and slot > 0:
                return
            origin = lax.rem(my_pos + SLOT_ORIGIN_DELTA[slot], N_DEV)
            wo_block = wo_ref[pl.ds(origin * HD, HD), :].astype(CDT)
            delta = jnp.dot(
                comm_ref[slot].reshape(B * Sq, HD), wo_block,
                preferred_element_type=jnp.float32,
            ).reshape(B, Sq, D_out)
            if slot == 0:
                out_ref[...] = delta
            else:
                out_ref[...] = out_ref[...] + delta

        def copy(src_at, dst_at, sem_idx, target):
            return pltpu.make_async_remote_copy(
                src_ref=src_at, dst_ref=dst_at,
                send_sem=send_sems.at[sem_idx],
                recv_sem=recv_sems.at[sem_idx],
                device_id=(target,), device_id_type=pl.DeviceIdType.MESH,
            )

        for b in range(B):
            for h in range(Hl // 2):
                attend(b, h)
        loA = (slice(None), slice(None), pl.ds(0, HALF))
        p1 = [
            copy(comm_ref.at[(0, *loA)], comm_ref.at[(1, *loA)], 0, right),
            copy(comm_ref.at[(0, *loA)], comm_ref.at[(2, *loA)], 1, left),
        ]
        p1[0].start()
        p1[1].start()

        for b in range(B):
            for h in range(Hl // 2, Hl):
                attend(b, h)
        hiA = (slice(None), slice(None), pl.ds(HALF, HALF))
        p1 += [
            copy(comm_ref.at[(0, *hiA)], comm_ref.at[(1, *hiA)], 2, right),
            copy(comm_ref.at[(0, *hiA)], comm_ref.at[(2, *hiA)], 3, left),
        ]
        p1[2].start()
        p1[3].start()
        contrib(0)

        p1[0].wait_recv()
        p1[2].wait_recv()
        p2_r = copy(comm_ref.at[1, 0], comm_ref.at[3, 0], 4, right)
        p2_r.start()
        contrib(1)
        p1[1].wait_recv()
        p1[3].wait_recv()
        p2_l = copy(comm_ref.at[2, 1], comm_ref.at[3, 1], 5, left)
        p2_l.start()
        contrib(2)
        p2_r.wait_recv()
        p2_l.wait_recv()
        contrib(3)
        for rdma in p1 + [p2_r, p2_l]:
            rdma.wait_send()

    return pl.pallas_call(
        body,
        out_shape=jax.ShapeDtypeStruct((B, Sq, D_out), jnp.float32),
        in_specs=[pl.BlockSpec(memory_space=pltpu.VMEM)] * 5,
        out_specs=pl.BlockSpec(memory_space=pltpu.VMEM),
        scratch_shapes=[
            pltpu.VMEM((N_DEV, B, Sq, HD), CDT),
            pltpu.SemaphoreType.DMA((6,)),
            pltpu.SemaphoreType.DMA((6,)),
        ],
        compiler_params=pltpu.CompilerParams(collective_id=0),
    )(x, Wq, K_ext, V_ext, Wo)
